# baseline (speedup 1.0000x reference)
import functools

import jax
import jax.numpy as jnp
from jax.experimental import pallas as pl
from jax.experimental.pallas import tpu as pltpu

TOKENS = 16384
D_MODEL = 4096
N_EXPERTS = 64
TOP_K = 2
TB = 1024
NSPLIT = 8
TBS = TB // NSPLIT


def _gate_kernel(*refs):
    x_refs = refs[:NSPLIT]
    w_ref, vals_ref, idx_ref, z_ref = refs[NSPLIT:]
    w = w_ref[...]
    dots = []
    for x_ref in x_refs:
        dots.append(jax.lax.dot_general(
            x_ref[...], w,
            dimension_numbers=(((1,), (1,)), ((), ())),
            preferred_element_type=jnp.float32,
            precision=jax.lax.Precision.DEFAULT,
        ))
    logits = jnp.concatenate(dots, axis=0)
    m = jnp.max(logits, axis=-1, keepdims=True)
    e = jnp.exp(logits - m)
    probs = e / jnp.sum(e, axis=-1, keepdims=True)

    iota = jax.lax.broadcasted_iota(jnp.int32, probs.shape, 1)
    v1 = jnp.max(probs, axis=-1, keepdims=True)
    i1 = jnp.min(jnp.where(probs == v1, iota, N_EXPERTS), axis=-1, keepdims=True)
    masked = jnp.where(iota == i1, -jnp.inf, probs)
    v2 = jnp.max(masked, axis=-1, keepdims=True)
    i2 = jnp.min(jnp.where(masked == v2, iota, N_EXPERTS), axis=-1, keepdims=True)

    vals_ref[...] = jnp.concatenate([v1, v2], axis=1).T
    idx_ref[...] = jnp.concatenate([i1, i2], axis=1).T
    z_ref[...] = jnp.zeros_like(z_ref)


@functools.partial(jax.jit, static_argnames=())
def _gate(x, w):
    grid = (TOKENS // TB,)
    vals, idx, z = pl.pallas_call(
        _gate_kernel,
        grid=grid,
        in_specs=[
            pl.BlockSpec((TBS, D_MODEL),
                         functools.partial(lambda s, i: (NSPLIT * i + s, 0), s))
            for s in range(NSPLIT)
        ] + [
            pl.BlockSpec((N_EXPERTS, D_MODEL), lambda i: (0, 0)),
        ],
        out_specs=[
            pl.BlockSpec((TOP_K, TB), lambda i: (0, i)),
            pl.BlockSpec((TOP_K, TB), lambda i: (0, i)),
            pl.BlockSpec((N_EXPERTS,), lambda i: (0,)),
        ],
        out_shape=[
            jax.ShapeDtypeStruct((TOP_K, TOKENS), jnp.float32),
            jax.ShapeDtypeStruct((TOP_K, TOKENS), jnp.int32),
            jax.ShapeDtypeStruct((N_EXPERTS,), jnp.float32),
        ],
        compiler_params=pltpu.CompilerParams(
            dimension_semantics=("parallel",),
        ),
    )(*([x] * NSPLIT), w)
    return vals.T, idx.T, z


def kernel(input, wg_weight):
    vals, idx, z = _gate(input, wg_weight)
    aux_loss = jnp.array(0.0, dtype=jnp.float32)
    return (aux_loss, vals, idx, z)


# speedup vs baseline: 1.0083x; 1.0083x over previous
import functools

import jax
import jax.numpy as jnp
from jax.experimental import pallas as pl
from jax.experimental.pallas import tpu as pltpu

TOKENS = 16384
D_MODEL = 4096
N_EXPERTS = 64
TOP_K = 2
TB = 1024
NSPLIT = 4
TBS = TB // NSPLIT


def _gate_kernel(*refs):
    x_refs = refs[:NSPLIT]
    w_ref, vals_ref, idx_ref, z_ref = refs[NSPLIT:]
    w = w_ref[...]
    dots = []
    for x_ref in x_refs:
        dots.append(jax.lax.dot_general(
            x_ref[...], w,
            dimension_numbers=(((1,), (1,)), ((), ())),
            preferred_element_type=jnp.float32,
            precision=jax.lax.Precision.DEFAULT,
        ))
    logits = jnp.concatenate(dots, axis=0)
    m = jnp.max(logits, axis=-1, keepdims=True)
    e = jnp.exp(logits - m)
    probs = e / jnp.sum(e, axis=-1, keepdims=True)

    iota = jax.lax.broadcasted_iota(jnp.int32, probs.shape, 1)
    v1 = jnp.max(probs, axis=-1, keepdims=True)
    i1 = jnp.min(jnp.where(probs == v1, iota, N_EXPERTS), axis=-1, keepdims=True)
    masked = jnp.where(iota == i1, -jnp.inf, probs)
    v2 = jnp.max(masked, axis=-1, keepdims=True)
    i2 = jnp.min(jnp.where(masked == v2, iota, N_EXPERTS), axis=-1, keepdims=True)

    vals_ref[...] = jnp.concatenate([v1, v2], axis=1).T
    idx_ref[...] = jnp.concatenate([i1, i2], axis=1).T
    z_ref[...] = jnp.zeros_like(z_ref)


@functools.partial(jax.jit, static_argnames=())
def _gate(x, w):
    grid = (TOKENS // TB,)
    vals, idx, z = pl.pallas_call(
        _gate_kernel,
        grid=grid,
        in_specs=[
            pl.BlockSpec((TBS, D_MODEL),
                         functools.partial(lambda s, i: (NSPLIT * i + s, 0), s))
            for s in range(NSPLIT)
        ] + [
            pl.BlockSpec((N_EXPERTS, D_MODEL), lambda i: (0, 0)),
        ],
        out_specs=[
            pl.BlockSpec((TOP_K, TB), lambda i: (0, i)),
            pl.BlockSpec((TOP_K, TB), lambda i: (0, i)),
            pl.BlockSpec((N_EXPERTS,), lambda i: (0,)),
        ],
        out_shape=[
            jax.ShapeDtypeStruct((TOP_K, TOKENS), jnp.float32),
            jax.ShapeDtypeStruct((TOP_K, TOKENS), jnp.int32),
            jax.ShapeDtypeStruct((N_EXPERTS,), jnp.float32),
        ],
        compiler_params=pltpu.CompilerParams(
            dimension_semantics=("parallel",),
        ),
    )(*([x] * NSPLIT), w)
    return vals.T, idx.T, z


def kernel(input, wg_weight):
    vals, idx, z = _gate(input, wg_weight)
    aux_loss = jnp.array(0.0, dtype=jnp.float32)
    return (aux_loss, vals, idx, z)
